# Initial kernel scaffold; baseline (speedup 1.0000x reference)
import jax
import jax.numpy as jnp
from jax.experimental import pallas as pl


def kernel(Gu, Gi, edge_row, edge_col, edge_vals, user, item):
    raise NotImplementedError("write your pallas kernel here")


# speedup vs baseline: 3.1767x; 3.1767x over previous
import jax
import jax.numpy as jnp
from jax import lax
from jax.experimental import pallas as pl
from jax.experimental.pallas import tpu as pltpu
from jax.experimental.pallas import tpu_sc as plsc

NUM_USERS = 5000
NUM_ITEMS = 5000
EMBED_K = 256
N_LAYERS = 3
N_EDGES = 160000
BATCH = 4096
N_NODES = NUM_USERS + NUM_ITEMS

NC = 2
NT = 16
HK = EMBED_K // NC
EPT = N_EDGES // NT
C = 80
NB = 25
NBLK = EPT // (C * NB)
BPT = BATCH // NT
GCH = 64
NGC = BPT // GCH
Q = HK // 16
RCH = 128
NFULL = N_NODES // RCH
TAIL = N_NODES - NFULL * RCH
ZR = 32


def _body(ego, er4, ec4, ev4, user, item,
          gu_out, gi_out, xui_out, scr0, scr1,
          rowb, colb, valb, rows, gtmp, gtmp2, zbuf,
          ubuf, ibuf, xvm, accum, sem):
  c = lax.axis_index("c")
  s = lax.axis_index("s")

  def zero_row(r, _):
    for q in range(Q):
      zbuf[r, pl.ds(q * 16, 16)] = jnp.zeros((16,), jnp.float32)
    return 0
  lax.fori_loop(0, ZR, zero_row, 0)

  def zero_accum_chunk(m):
    base = m * RCH
    for h in range(RCH // ZR):
      pltpu.sync_copy(zbuf, accum.at[pl.ds(base + h * ZR, ZR)])

  def tail_zero():
    pltpu.sync_copy(zbuf.at[pl.ds(0, TAIL)],
                    accum.at[pl.ds(NFULL * RCH, TAIL)])

  for i in range(5):
    m = s + NT * i
    @pl.when(m < NFULL)
    def _():
      zero_accum_chunk(m)
  @pl.when(s == NT - 1)
  def _():
    tail_zero()

  pltpu.sync_copy(user.at[pl.ds(s * BPT, BPT)], ubuf)
  pltpu.sync_copy(item.at[pl.ds(s * BPT, BPT)], ibuf)
  def shift_item(q, _):
    ibuf[pl.ds(q * 16, 16)] = ibuf[pl.ds(q * 16, 16)] + NUM_USERS
    return 0
  lax.fori_loop(0, BPT // 16, shift_item, 0)

  def gamma_accum(src_ref, alpha, init):
    for idxbuf, out in ((ubuf, gu_out), (ibuf, gi_out)):
      for j in range(NGC):
        osl = pl.ds(s * BPT + j * GCH, GCH)
        pltpu.async_copy(src_ref.at[idxbuf.at[pl.ds(j * GCH, GCH)]],
                         gtmp, sem).wait()
        if not init:
          pltpu.sync_copy(out.at[c].at[osl], gtmp2)
        def upd(r, _):
          for q in range(Q):
            sl = pl.ds(q * 16, 16)
            v = gtmp[r, sl] * alpha
            if not init:
              v = v + gtmp2[r, sl]
            gtmp[r, sl] = v
          return 0
        lax.fori_loop(0, GCH, upd, 0)
        pltpu.sync_copy(gtmp, out.at[c].at[osl])

  gamma_accum(ego.at[c], 0.25, True)

  srcs = (ego, scr0, scr1)
  dsts = (scr0, scr1, scr0)
  for k in range(1, N_LAYERS + 1):
    src = srcs[k - 1].at[c]
    dst = dsts[k - 1].at[c]
    plsc.subcore_barrier()

    def edge_block(b, _):
      pltpu.sync_copy(er4.at[s].at[b], rowb)
      pltpu.sync_copy(ec4.at[s].at[b], colb)
      pltpu.sync_copy(ev4.at[s].at[b], valb)
      def edge_chunk(j, _):
        pltpu.async_copy(src.at[colb.at[j]], rows, sem).wait()
        jv = jnp.full((16,), j, jnp.int32)
        def scale(e, _):
          v = plsc.load_gather(valb, [jv, jnp.full((16,), e, jnp.int32)])
          for q in range(Q):
            sl = pl.ds(q * 16, 16)
            rows[e, sl] = rows[e, sl] * v
          return 0
        lax.fori_loop(0, C, scale, 0)
        pltpu.sync_copy(rows, accum.at[rowb.at[j]], add=True)
        return 0
      lax.fori_loop(0, NB, edge_chunk, 0)
      return 0
    lax.fori_loop(0, NBLK, edge_block, 0)

    plsc.subcore_barrier()

    for i in range(5):
      m = s + NT * i
      @pl.when(m < NFULL)
      def _():
        for h in range(RCH // GCH):
          rsl = pl.ds(m * RCH + h * GCH, GCH)
          pltpu.sync_copy(accum.at[rsl], gtmp)
          pltpu.sync_copy(gtmp, dst.at[rsl])
        if k < N_LAYERS:
          zero_accum_chunk(m)
    @pl.when(s == NT - 1)
    def _():
      tsl = pl.ds(NFULL * RCH, TAIL)
      pltpu.sync_copy(accum.at[tsl], gtmp.at[pl.ds(0, TAIL)])
      pltpu.sync_copy(gtmp.at[pl.ds(0, TAIL)], dst.at[tsl])
      if k < N_LAYERS:
        tail_zero()

    plsc.subcore_barrier()

    gamma_accum(dst, 0.25 / (1.0 + k), False)

  for j in range(NGC):
    osl = pl.ds(s * BPT + j * GCH, GCH)
    pltpu.sync_copy(gu_out.at[c].at[osl], gtmp)
    pltpu.sync_copy(gi_out.at[c].at[osl], gtmp2)
    def dot_group(g, _):
      row_ids = g * 16 + lax.iota(jnp.int32, 16)
      def dot_dim(d, acc):
        col = jnp.full((16,), d, jnp.int32)
        u = plsc.load_gather(gtmp, [row_ids, col])
        v = plsc.load_gather(gtmp2, [row_ids, col])
        return acc + u * v
      acc = lax.fori_loop(0, HK, dot_dim, jnp.zeros((16,), jnp.float32))
      xvm[pl.ds(j * GCH + g * 16, 16)] = acc
      return 0
    lax.fori_loop(0, GCH // 16, dot_group, 0)
  pltpu.sync_copy(xvm, xui_out.at[c].at[pl.ds(s * BPT, BPT)])


@jax.jit
def _run(ego_split, er4, ec4, ev4, user, item):
  f32 = jnp.float32
  kern = pl.kernel(
      _body,
      out_type=(
          jax.ShapeDtypeStruct((NC, BATCH, HK), f32),
          jax.ShapeDtypeStruct((NC, BATCH, HK), f32),
          jax.ShapeDtypeStruct((NC, BATCH), f32),
          jax.ShapeDtypeStruct((NC, N_NODES, HK), f32),
          jax.ShapeDtypeStruct((NC, N_NODES, HK), f32),
      ),
      mesh=plsc.VectorSubcoreMesh(core_axis_name="c", subcore_axis_name="s"),
      compiler_params=pltpu.CompilerParams(needs_layout_passes=False),
      scratch_types=[
          pltpu.VMEM((NB, C), jnp.int32),
          pltpu.VMEM((NB, C), jnp.int32),
          pltpu.VMEM((NB, C), f32),
          pltpu.VMEM((C, HK), f32),
          pltpu.VMEM((GCH, HK), f32),
          pltpu.VMEM((GCH, HK), f32),
          pltpu.VMEM((ZR, HK), f32),
          pltpu.VMEM((BPT,), jnp.int32),
          pltpu.VMEM((BPT,), jnp.int32),
          pltpu.VMEM((BPT,), f32),
          pltpu.VMEM_SHARED((N_NODES, HK), f32),
          pltpu.SemaphoreType.DMA,
      ],
  )
  return kern(ego_split, er4, ec4, ev4, user, item)


def kernel(Gu, Gi, edge_row, edge_col, edge_vals, user, item):
  ego = jnp.concatenate([Gu, Gi], axis=0)
  ego_split = jnp.stack([ego[:, :HK], ego[:, HK:]])
  er4 = edge_row.reshape(NT, NBLK, NB, C)
  ec4 = edge_col.reshape(NT, NBLK, NB, C)
  ev4 = edge_vals.reshape(NT, NBLK, NB, C)
  gu, gi, xui, _, _ = _run(ego_split, er4, ec4, ev4, user, item)
  gamma_u = jnp.concatenate([gu[0], gu[1]], axis=1)
  gamma_i = jnp.concatenate([gi[0], gi[1]], axis=1)
  return (xui[0] + xui[1], gamma_u, gamma_i)


# speedup vs baseline: 4.7712x; 1.5019x over previous
import jax
import jax.numpy as jnp
from jax import lax
from jax.experimental import pallas as pl
from jax.experimental.pallas import tpu as pltpu
from jax.experimental.pallas import tpu_sc as plsc

NUM_USERS = 5000
NUM_ITEMS = 5000
EMBED_K = 256
N_LAYERS = 3
N_EDGES = 160000
BATCH = 4096
N_NODES = NUM_USERS + NUM_ITEMS

NC = 2
NT = 16
HK = EMBED_K // NC
EPT = N_EDGES // NT
C = 100
NB = 20
NBLK = EPT // (C * NB)
BPT = BATCH // NT
GCH = 32
OCH = 64
NGC = BPT // GCH
Q = HK // 16
RCH = 128
NFULL = N_NODES // RCH
TAIL = N_NODES - NFULL * RCH
ZR = 16


def _body(ego, er4, ec4, ev4, user, item,
          gu_out, gi_out, xui_out, scr0, scr1,
          rowb, colb, valb, rows0, rows1, gtmp, gtmp2, zbuf,
          ubuf, ibuf, xvm, accum, gsem, ssem):
  c = lax.axis_index("c")
  s = lax.axis_index("s")

  def zero_row(r, _):
    for q in range(Q):
      zbuf[r, pl.ds(q * 16, 16)] = jnp.zeros((16,), jnp.float32)
    return 0
  lax.fori_loop(0, ZR, zero_row, 0)

  def zero_accum_chunk(m):
    base = m * RCH
    for h in range(RCH // ZR):
      pltpu.async_copy(zbuf, accum.at[pl.ds(base + h * ZR, ZR)], gsem)
    for h in range(RCH // ZR):
      pltpu.make_async_copy(zbuf, accum.at[pl.ds(base + h * ZR, ZR)],
                            gsem).wait()

  def tail_zero():
    pltpu.sync_copy(zbuf.at[pl.ds(0, TAIL)],
                    accum.at[pl.ds(NFULL * RCH, TAIL)])

  for i in range(5):
    m = s + NT * i
    @pl.when(m < NFULL)
    def _():
      zero_accum_chunk(m)
  @pl.when(s == NT - 1)
  def _():
    tail_zero()

  pltpu.sync_copy(user.at[pl.ds(s * BPT, BPT)], ubuf)
  pltpu.sync_copy(item.at[pl.ds(s * BPT, BPT)], ibuf)
  def shift_item(q, _):
    ibuf[pl.ds(q * 16, 16)] = ibuf[pl.ds(q * 16, 16)] + NUM_USERS
    return 0
  lax.fori_loop(0, BPT // 16, shift_item, 0)

  def gamma_accum(src_ref, alpha, init):
    for idxbuf, out in ((ubuf, gu_out), (ibuf, gi_out)):
      for j in range(NGC):
        osl = pl.ds(s * BPT + j * GCH, GCH)
        pltpu.async_copy(src_ref.at[idxbuf.at[pl.ds(j * GCH, GCH)]],
                         gtmp, gsem).wait()
        if not init:
          pltpu.sync_copy(out.at[c].at[osl], gtmp2)
        def upd(r, _):
          for q in range(Q):
            sl = pl.ds(q * 16, 16)
            v = gtmp[r, sl] * alpha
            if not init:
              v = v + gtmp2[r, sl]
            gtmp[r, sl] = v
          return 0
        lax.fori_loop(0, GCH, upd, 0)
        pltpu.sync_copy(gtmp, out.at[c].at[osl])

  gamma_accum(ego.at[c], 0.25, True)

  def scale_chunk(buf, j):
    jv = jnp.full((16,), j, jnp.int32)
    def scale(e2, _):
      for u in range(2):
        e = e2 * 2 + u
        v = plsc.load_gather(valb, [jv, jnp.full((16,), e, jnp.int32)])
        for q in range(Q):
          sl = pl.ds(q * 16, 16)
          buf[e, sl] = buf[e, sl] * v
      return 0
    lax.fori_loop(0, C // 2, scale, 0)

  srcs = (ego, scr0, scr1)
  dsts = (scr0, scr1, scr0)
  for k in range(1, N_LAYERS + 1):
    src = srcs[k - 1].at[c]
    dst = dsts[k - 1].at[c]
    plsc.subcore_barrier()

    def edge_block(b, _):
      pltpu.sync_copy(er4.at[s].at[b], rowb)
      pltpu.sync_copy(ec4.at[s].at[b], colb)
      pltpu.sync_copy(ev4.at[s].at[b], valb)

      pltpu.async_copy(src.at[colb.at[0]], rows0, gsem)

      def step(j, cur, nxt):
        pltpu.make_async_copy(src.at[colb.at[j]], cur, gsem).wait()
        @pl.when(j >= 1)
        def _():
          pltpu.make_async_copy(nxt, accum.at[rowb.at[j - 1]], ssem).wait()
        @pl.when(j + 1 < NB)
        def _():
          pltpu.async_copy(src.at[colb.at[j + 1]], nxt, gsem)
        scale_chunk(cur, j)
        pltpu.async_copy(cur, accum.at[rowb.at[j]], ssem, add=True)

      def pair(p, _):
        step(2 * p, rows0, rows1)
        step(2 * p + 1, rows1, rows0)
        return 0
      lax.fori_loop(0, NB // 2, pair, 0)
      pltpu.make_async_copy(rows1, accum.at[rowb.at[NB - 1]], ssem).wait()
      return 0
    lax.fori_loop(0, NBLK, edge_block, 0)

    plsc.subcore_barrier()

    for i in range(5):
      m = s + NT * i
      @pl.when(m < NFULL)
      def _():
        for h, buf in ((0, rows0), (1, rows1)):
          rsl = pl.ds(m * RCH + h * OCH, OCH)
          pltpu.sync_copy(accum.at[rsl], buf.at[pl.ds(0, OCH)])
          pltpu.async_copy(buf.at[pl.ds(0, OCH)], dst.at[rsl], ssem)
        for h, buf in ((0, rows0), (1, rows1)):
          rsl = pl.ds(m * RCH + h * OCH, OCH)
          pltpu.make_async_copy(buf.at[pl.ds(0, OCH)], dst.at[rsl],
                                ssem).wait()
        if k < N_LAYERS:
          zero_accum_chunk(m)
    @pl.when(s == NT - 1)
    def _():
      tsl = pl.ds(NFULL * RCH, TAIL)
      pltpu.sync_copy(accum.at[tsl], gtmp.at[pl.ds(0, TAIL)])
      pltpu.sync_copy(gtmp.at[pl.ds(0, TAIL)], dst.at[tsl])
      if k < N_LAYERS:
        tail_zero()

    plsc.subcore_barrier()

    gamma_accum(dst, 0.25 / (1.0 + k), False)

  for j in range(NGC):
    osl = pl.ds(s * BPT + j * GCH, GCH)
    pltpu.sync_copy(gu_out.at[c].at[osl], gtmp)
    pltpu.sync_copy(gi_out.at[c].at[osl], gtmp2)
    def dot_group(g, _):
      row_ids = g * 16 + lax.iota(jnp.int32, 16)
      def dot_dim(d, acc):
        col = jnp.full((16,), d, jnp.int32)
        u = plsc.load_gather(gtmp, [row_ids, col])
        v = plsc.load_gather(gtmp2, [row_ids, col])
        return acc + u * v
      acc = lax.fori_loop(0, HK, dot_dim, jnp.zeros((16,), jnp.float32))
      xvm[pl.ds(j * GCH + g * 16, 16)] = acc
      return 0
    lax.fori_loop(0, GCH // 16, dot_group, 0)
  pltpu.sync_copy(xvm, xui_out.at[c].at[pl.ds(s * BPT, BPT)])


@jax.jit
def _run(ego_split, er4, ec4, ev4, user, item):
  f32 = jnp.float32
  kern = pl.kernel(
      _body,
      out_type=(
          jax.ShapeDtypeStruct((NC, BATCH, HK), f32),
          jax.ShapeDtypeStruct((NC, BATCH, HK), f32),
          jax.ShapeDtypeStruct((NC, BATCH), f32),
          jax.ShapeDtypeStruct((NC, N_NODES, HK), f32),
          jax.ShapeDtypeStruct((NC, N_NODES, HK), f32),
      ),
      mesh=plsc.VectorSubcoreMesh(core_axis_name="c", subcore_axis_name="s"),
      compiler_params=pltpu.CompilerParams(needs_layout_passes=False),
      scratch_types=[
          pltpu.VMEM((NB, C), jnp.int32),
          pltpu.VMEM((NB, C), jnp.int32),
          pltpu.VMEM((NB, C), f32),
          pltpu.VMEM((C, HK), f32),
          pltpu.VMEM((C, HK), f32),
          pltpu.VMEM((GCH, HK), f32),
          pltpu.VMEM((GCH, HK), f32),
          pltpu.VMEM((ZR, HK), f32),
          pltpu.VMEM((BPT,), jnp.int32),
          pltpu.VMEM((BPT,), jnp.int32),
          pltpu.VMEM((BPT,), f32),
          pltpu.VMEM_SHARED((N_NODES, HK), f32),
          pltpu.SemaphoreType.DMA,
          pltpu.SemaphoreType.DMA,
      ],
  )
  return kern(ego_split, er4, ec4, ev4, user, item)


def kernel(Gu, Gi, edge_row, edge_col, edge_vals, user, item):
  ego = jnp.concatenate([Gu, Gi], axis=0)
  ego_split = jnp.stack([ego[:, :HK], ego[:, HK:]])
  er4 = edge_row.reshape(NT, NBLK, NB, C)
  ec4 = edge_col.reshape(NT, NBLK, NB, C)
  ev4 = edge_vals.reshape(NT, NBLK, NB, C)
  gu, gi, xui, _, _ = _run(ego_split, er4, ec4, ev4, user, item)
  gamma_u = jnp.concatenate([gu[0], gu[1]], axis=1)
  gamma_i = jnp.concatenate([gi[0], gi[1]], axis=1)
  return (xui[0] + xui[1], gamma_u, gamma_i)


# speedup vs baseline: 4.8492x; 1.0164x over previous
import jax
import jax.numpy as jnp
from jax import lax
from jax.experimental import pallas as pl
from jax.experimental.pallas import tpu as pltpu
from jax.experimental.pallas import tpu_sc as plsc

NUM_USERS = 5000
NUM_ITEMS = 5000
EMBED_K = 256
N_LAYERS = 3
N_EDGES = 160000
BATCH = 4096
N_NODES = NUM_USERS + NUM_ITEMS

NC = 2
NT = 16
HK = EMBED_K // NC
EPT = N_EDGES // NT
C = 125
NB = 20
NBLK = EPT // (C * NB)
BPT = BATCH // NT
GCH = 32
OCH = 64
NGC = BPT // GCH
Q = HK // 16
RCH = 128
NFULL = N_NODES // RCH
TAIL = N_NODES - NFULL * RCH
ZR = 16


def _body(ego, er4, ec4, ev4, user, item,
          gu_out, gi_out, xui_out, scr0, scr1,
          rowb, colb, valb, rows0, rows1, zbuf,
          ubuf, ibuf, xvm, accum, gsem, gsem1, ssem):
  gtmp = rows0
  gtmp2 = rows1
  c = lax.axis_index("c")
  s = lax.axis_index("s")

  def zero_row(r, _):
    for q in range(Q):
      zbuf[r, pl.ds(q * 16, 16)] = jnp.zeros((16,), jnp.float32)
    return 0
  lax.fori_loop(0, ZR, zero_row, 0)

  def zero_accum_chunk(m):
    base = m * RCH
    for h in range(RCH // ZR):
      pltpu.async_copy(zbuf, accum.at[pl.ds(base + h * ZR, ZR)], gsem)
    for h in range(RCH // ZR):
      pltpu.make_async_copy(zbuf, accum.at[pl.ds(base + h * ZR, ZR)],
                            gsem).wait()

  def tail_zero():
    pltpu.sync_copy(zbuf.at[pl.ds(0, TAIL)],
                    accum.at[pl.ds(NFULL * RCH, TAIL)])

  for i in range(5):
    m = s + NT * i
    @pl.when(m < NFULL)
    def _():
      zero_accum_chunk(m)
  @pl.when(s == NT - 1)
  def _():
    tail_zero()

  pltpu.sync_copy(user.at[pl.ds(s * BPT, BPT)], ubuf)
  pltpu.sync_copy(item.at[pl.ds(s * BPT, BPT)], ibuf)
  def shift_item(q, _):
    ibuf[pl.ds(q * 16, 16)] = ibuf[pl.ds(q * 16, 16)] + NUM_USERS
    return 0
  lax.fori_loop(0, BPT // 16, shift_item, 0)

  def gamma_accum(src_ref, alpha, init):
    for idxbuf, out in ((ubuf, gu_out), (ibuf, gi_out)):
      for j in range(NGC):
        osl = pl.ds(s * BPT + j * GCH, GCH)
        pltpu.async_copy(src_ref.at[idxbuf.at[pl.ds(j * GCH, GCH)]],
                         gtmp.at[pl.ds(0, GCH)], gsem).wait()
        if not init:
          pltpu.sync_copy(out.at[c].at[osl], gtmp2.at[pl.ds(0, GCH)])
        def upd(r, _):
          for q in range(Q):
            sl = pl.ds(q * 16, 16)
            v = gtmp[r, sl] * alpha
            if not init:
              v = v + gtmp2[r, sl]
            gtmp[r, sl] = v
          return 0
        lax.fori_loop(0, GCH, upd, 0)
        pltpu.sync_copy(gtmp.at[pl.ds(0, GCH)], out.at[c].at[osl])

  gamma_accum(ego.at[c], 0.25, True)

  one = jnp.full((16,), 1, jnp.int32)

  def scale_chunk(buf, j):
    jv = jnp.full((16,), j, jnp.int32)
    def scale(e2, ev):
      for u in range(2):
        e = e2 * 2 + u
        v = plsc.load_gather(valb, [jv, ev])
        ev = ev + one
        for q in range(Q):
          sl = pl.ds(q * 16, 16)
          buf[e, sl] = buf[e, sl] * v
      return ev
    ev = lax.fori_loop(0, C // 2, scale, jnp.zeros((16,), jnp.int32))
    if C % 2:
      v = plsc.load_gather(valb, [jv, ev])
      for q in range(Q):
        sl = pl.ds(q * 16, 16)
        buf[C - 1, sl] = buf[C - 1, sl] * v

  srcs = (ego, scr0, scr1)
  dsts = (scr0, scr1, scr0)
  for k in range(1, N_LAYERS + 1):
    src = srcs[k - 1].at[c]
    dst = dsts[k - 1].at[c]
    plsc.subcore_barrier()

    def edge_block(b, _):
      pltpu.sync_copy(er4.at[s].at[b], rowb)
      pltpu.sync_copy(ec4.at[s].at[b], colb)
      pltpu.sync_copy(ev4.at[s].at[b], valb)

      pltpu.async_copy(src.at[colb.at[0]], rows0, gsem)

      def step(j, cur, nxt, sem_cur, sem_nxt):
        @pl.when(j >= 1)
        def _():
          pltpu.make_async_copy(nxt, accum.at[rowb.at[j - 1]], ssem).wait()
        @pl.when(j + 1 < NB)
        def _():
          pltpu.async_copy(src.at[colb.at[j + 1]], nxt, sem_nxt)
        pltpu.make_async_copy(src.at[colb.at[j]], cur, sem_cur).wait()
        scale_chunk(cur, j)
        pltpu.async_copy(cur, accum.at[rowb.at[j]], ssem, add=True)

      def pair(p, _):
        step(2 * p, rows0, rows1, gsem, gsem1)
        step(2 * p + 1, rows1, rows0, gsem1, gsem)
        return 0
      lax.fori_loop(0, NB // 2, pair, 0)
      pltpu.make_async_copy(rows1, accum.at[rowb.at[NB - 1]], ssem).wait()
      return 0
    lax.fori_loop(0, NBLK, edge_block, 0)

    plsc.subcore_barrier()

    for i in range(5):
      m = s + NT * i
      @pl.when(m < NFULL)
      def _():
        for h, buf in ((0, rows0), (1, rows1)):
          rsl = pl.ds(m * RCH + h * OCH, OCH)
          pltpu.sync_copy(accum.at[rsl], buf.at[pl.ds(0, OCH)])
          pltpu.async_copy(buf.at[pl.ds(0, OCH)], dst.at[rsl], ssem)
        for h, buf in ((0, rows0), (1, rows1)):
          rsl = pl.ds(m * RCH + h * OCH, OCH)
          pltpu.make_async_copy(buf.at[pl.ds(0, OCH)], dst.at[rsl],
                                ssem).wait()
        if k < N_LAYERS:
          zero_accum_chunk(m)
    @pl.when(s == NT - 1)
    def _():
      tsl = pl.ds(NFULL * RCH, TAIL)
      pltpu.sync_copy(accum.at[tsl], gtmp.at[pl.ds(0, TAIL)])
      pltpu.sync_copy(gtmp.at[pl.ds(0, TAIL)], dst.at[tsl])
      if k < N_LAYERS:
        tail_zero()

    plsc.subcore_barrier()

    gamma_accum(dst, 0.25 / (1.0 + k), False)

  for j in range(NGC):
    osl = pl.ds(s * BPT + j * GCH, GCH)
    pltpu.sync_copy(gu_out.at[c].at[osl], gtmp.at[pl.ds(0, GCH)])
    pltpu.sync_copy(gi_out.at[c].at[osl], gtmp2.at[pl.ds(0, GCH)])
    def dot_group(g, _):
      row_ids = g * 16 + lax.iota(jnp.int32, 16)
      def dot_dim(d, acc):
        col = jnp.full((16,), d, jnp.int32)
        u = plsc.load_gather(gtmp, [row_ids, col])
        v = plsc.load_gather(gtmp2, [row_ids, col])
        return acc + u * v
      acc = lax.fori_loop(0, HK, dot_dim, jnp.zeros((16,), jnp.float32))
      xvm[pl.ds(j * GCH + g * 16, 16)] = acc
      return 0
    lax.fori_loop(0, GCH // 16, dot_group, 0)
  pltpu.sync_copy(xvm, xui_out.at[c].at[pl.ds(s * BPT, BPT)])


@jax.jit
def _run(ego_split, er4, ec4, ev4, user, item):
  f32 = jnp.float32
  kern = pl.kernel(
      _body,
      out_type=(
          jax.ShapeDtypeStruct((NC, BATCH, HK), f32),
          jax.ShapeDtypeStruct((NC, BATCH, HK), f32),
          jax.ShapeDtypeStruct((NC, BATCH), f32),
          jax.ShapeDtypeStruct((NC, N_NODES, HK), f32),
          jax.ShapeDtypeStruct((NC, N_NODES, HK), f32),
      ),
      mesh=plsc.VectorSubcoreMesh(core_axis_name="c", subcore_axis_name="s"),
      compiler_params=pltpu.CompilerParams(needs_layout_passes=False),
      scratch_types=[
          pltpu.VMEM((NB, C), jnp.int32),
          pltpu.VMEM((NB, C), jnp.int32),
          pltpu.VMEM((NB, C), f32),
          pltpu.VMEM((C, HK), f32),
          pltpu.VMEM((C, HK), f32),
          pltpu.VMEM((ZR, HK), f32),
          pltpu.VMEM((BPT,), jnp.int32),
          pltpu.VMEM((BPT,), jnp.int32),
          pltpu.VMEM((BPT,), f32),
          pltpu.VMEM_SHARED((N_NODES, HK), f32),
          pltpu.SemaphoreType.DMA,
          pltpu.SemaphoreType.DMA,
          pltpu.SemaphoreType.DMA,
      ],
  )
  return kern(ego_split, er4, ec4, ev4, user, item)


def kernel(Gu, Gi, edge_row, edge_col, edge_vals, user, item):
  ego = jnp.concatenate([Gu, Gi], axis=0)
  ego_split = jnp.stack([ego[:, :HK], ego[:, HK:]])
  er4 = edge_row.reshape(NT, NBLK, NB, C)
  ec4 = edge_col.reshape(NT, NBLK, NB, C)
  ev4 = edge_vals.reshape(NT, NBLK, NB, C)
  gu, gi, xui, _, _ = _run(ego_split, er4, ec4, ev4, user, item)
  gamma_u = jnp.concatenate([gu[0], gu[1]], axis=1)
  gamma_i = jnp.concatenate([gi[0], gi[1]], axis=1)
  return (xui[0] + xui[1], gamma_u, gamma_i)


# speedup vs baseline: 4.8578x; 1.0018x over previous
import jax
import jax.numpy as jnp
from jax import lax
from jax.experimental import pallas as pl
from jax.experimental.pallas import tpu as pltpu
from jax.experimental.pallas import tpu_sc as plsc

NUM_USERS = 5000
NUM_ITEMS = 5000
EMBED_K = 256
N_LAYERS = 3
N_EDGES = 160000
BATCH = 4096
N_NODES = NUM_USERS + NUM_ITEMS

NC = 2
NT = 16
HK = EMBED_K // NC
EPT = N_EDGES // NT
C = 125
NB = 20
NBLK = EPT // (C * NB)
BPT = BATCH // NT
GCH = 32
OCH = 64
NGC = BPT // GCH
Q = HK // 16
RCH = 128
NFULL = N_NODES // RCH
TAIL = N_NODES - NFULL * RCH
ZR = 16


def _body(ego, er4, ec4, ev3, user, item,
          gu_out, gi_out, xui_out, scr0, scr1,
          rowb, colb, valb, rows0, rows1, zbuf,
          ubuf, ibuf, xvm, accum, gsem, gsem1, ssem):
  gtmp = rows0
  gtmp2 = rows1
  c = lax.axis_index("c")
  s = lax.axis_index("s")

  def zero_row(r, _):
    for q in range(Q):
      zbuf[r, pl.ds(q * 16, 16)] = jnp.zeros((16,), jnp.float32)
    return 0
  lax.fori_loop(0, ZR, zero_row, 0)

  def zero_accum_chunk(m):
    base = m * RCH
    for h in range(RCH // ZR):
      pltpu.async_copy(zbuf, accum.at[pl.ds(base + h * ZR, ZR)], gsem)
    for h in range(RCH // ZR):
      pltpu.make_async_copy(zbuf, accum.at[pl.ds(base + h * ZR, ZR)],
                            gsem).wait()

  def tail_zero():
    pltpu.sync_copy(zbuf.at[pl.ds(0, TAIL)],
                    accum.at[pl.ds(NFULL * RCH, TAIL)])

  for i in range(5):
    m = s + NT * i
    @pl.when(m < NFULL)
    def _():
      zero_accum_chunk(m)
  @pl.when(s == NT - 1)
  def _():
    tail_zero()

  pltpu.sync_copy(user.at[pl.ds(s * BPT, BPT)], ubuf)
  pltpu.sync_copy(item.at[pl.ds(s * BPT, BPT)], ibuf)
  def shift_item(q, _):
    ibuf[pl.ds(q * 16, 16)] = ibuf[pl.ds(q * 16, 16)] + NUM_USERS
    return 0
  lax.fori_loop(0, BPT // 16, shift_item, 0)

  def gamma_accum(src_ref, alpha, init):
    for idxbuf, out in ((ubuf, gu_out), (ibuf, gi_out)):
      for j in range(NGC):
        osl = pl.ds(s * BPT + j * GCH, GCH)
        pltpu.async_copy(src_ref.at[idxbuf.at[pl.ds(j * GCH, GCH)]],
                         gtmp.at[pl.ds(0, GCH)], gsem).wait()
        if not init:
          pltpu.sync_copy(out.at[c].at[osl], gtmp2.at[pl.ds(0, GCH)])
        def upd(r, _):
          for q in range(Q):
            sl = pl.ds(q * 16, 16)
            v = gtmp[r, sl] * alpha
            if not init:
              v = v + gtmp2[r, sl]
            gtmp[r, sl] = v
          return 0
        lax.fori_loop(0, GCH, upd, 0)
        pltpu.sync_copy(gtmp.at[pl.ds(0, GCH)], out.at[c].at[osl])

  with jax.named_scope("gamma0"):
    gamma_accum(ego.at[c], 0.25, True)

  one = jnp.full((16,), 1, jnp.int32)

  def scale_chunk(buf, j):
    def scale(e2, ev):
      for u in range(2):
        e = e2 * 2 + u
        v = plsc.load_gather(valb, [ev])
        ev = ev + one
        for q in range(Q):
          sl = pl.ds(q * 16, 16)
          buf[e, sl] = buf[e, sl] * v
      return ev
    ev0 = jnp.full((16,), j * C, jnp.int32)
    ev = lax.fori_loop(0, C // 2, scale, ev0)
    if C % 2:
      v = plsc.load_gather(valb, [ev])
      for q in range(Q):
        sl = pl.ds(q * 16, 16)
        buf[C - 1, sl] = buf[C - 1, sl] * v

  srcs = (ego, scr0, scr1)
  dsts = (scr0, scr1, scr0)
  for k in range(1, N_LAYERS + 1):
    src = srcs[k - 1].at[c]
    dst = dsts[k - 1].at[c]
    plsc.subcore_barrier()

    def edge_block(b, _, src=src):
      pltpu.sync_copy(er4.at[s].at[b], rowb)
      pltpu.sync_copy(ec4.at[s].at[b], colb)
      pltpu.sync_copy(ev3.at[s].at[b], valb)

      pltpu.async_copy(src.at[colb.at[0]], rows0, gsem)

      def step(j, cur, nxt, sem_cur, sem_nxt):
        @pl.when(j >= 1)
        def _():
          pltpu.make_async_copy(nxt, accum.at[rowb.at[j - 1]], ssem).wait()
        @pl.when(j + 1 < NB)
        def _():
          pltpu.async_copy(src.at[colb.at[j + 1]], nxt, sem_nxt)
        pltpu.make_async_copy(src.at[colb.at[j]], cur, sem_cur).wait()
        scale_chunk(cur, j)
        pltpu.async_copy(cur, accum.at[rowb.at[j]], ssem, add=True)

      def pair(p, _):
        step(2 * p, rows0, rows1, gsem, gsem1)
        step(2 * p + 1, rows1, rows0, gsem1, gsem)
        return 0
      lax.fori_loop(0, NB // 2, pair, 0)
      pltpu.make_async_copy(rows1, accum.at[rowb.at[NB - 1]], ssem).wait()
      return 0
    with jax.named_scope(f"edges{k}"):
      lax.fori_loop(0, NBLK, edge_block, 0)
      plsc.subcore_barrier()

    with jax.named_scope(f"copyout{k}"):
      for i in range(5):
        m = s + NT * i
        @pl.when(m < NFULL)
        def _():
          for h, buf in ((0, rows0), (1, rows1)):
            rsl = pl.ds(m * RCH + h * OCH, OCH)
            pltpu.sync_copy(accum.at[rsl], buf.at[pl.ds(0, OCH)])
            pltpu.async_copy(buf.at[pl.ds(0, OCH)], dst.at[rsl], ssem)
          for h, buf in ((0, rows0), (1, rows1)):
            rsl = pl.ds(m * RCH + h * OCH, OCH)
            pltpu.make_async_copy(buf.at[pl.ds(0, OCH)], dst.at[rsl],
                                  ssem).wait()
          if k < N_LAYERS:
            zero_accum_chunk(m)
      @pl.when(s == NT - 1)
      def _():
        tsl = pl.ds(NFULL * RCH, TAIL)
        pltpu.sync_copy(accum.at[tsl], gtmp.at[pl.ds(0, TAIL)])
        pltpu.sync_copy(gtmp.at[pl.ds(0, TAIL)], dst.at[tsl])
        if k < N_LAYERS:
          tail_zero()

      plsc.subcore_barrier()

    with jax.named_scope(f"gamma{k}"):
      gamma_accum(dst, 0.25 / (1.0 + k), False)

  with jax.named_scope("dot"):
   for j in range(NGC):
    osl = pl.ds(s * BPT + j * GCH, GCH)
    pltpu.sync_copy(gu_out.at[c].at[osl], gtmp.at[pl.ds(0, GCH)])
    pltpu.sync_copy(gi_out.at[c].at[osl], gtmp2.at[pl.ds(0, GCH)])
    def dot_group(g, _):
      row_ids = g * 16 + lax.iota(jnp.int32, 16)
      def dot_dim(d, acc):
        col = jnp.full((16,), d, jnp.int32)
        u = plsc.load_gather(gtmp, [row_ids, col])
        v = plsc.load_gather(gtmp2, [row_ids, col])
        return acc + u * v
      acc = lax.fori_loop(0, HK, dot_dim, jnp.zeros((16,), jnp.float32))
      xvm[pl.ds(j * GCH + g * 16, 16)] = acc
      return 0
    lax.fori_loop(0, GCH // 16, dot_group, 0)
  pltpu.sync_copy(xvm, xui_out.at[c].at[pl.ds(s * BPT, BPT)])


@jax.jit
def _run(ego_split, er4, ec4, ev4, user, item):
  f32 = jnp.float32
  kern = pl.kernel(
      _body,
      out_type=(
          jax.ShapeDtypeStruct((NC, BATCH, HK), f32),
          jax.ShapeDtypeStruct((NC, BATCH, HK), f32),
          jax.ShapeDtypeStruct((NC, BATCH), f32),
          jax.ShapeDtypeStruct((NC, N_NODES, HK), f32),
          jax.ShapeDtypeStruct((NC, N_NODES, HK), f32),
      ),
      mesh=plsc.VectorSubcoreMesh(core_axis_name="c", subcore_axis_name="s"),
      compiler_params=pltpu.CompilerParams(needs_layout_passes=False),
      scratch_types=[
          pltpu.VMEM((NB, C), jnp.int32),
          pltpu.VMEM((NB, C), jnp.int32),
          pltpu.VMEM((NB * C,), f32),
          pltpu.VMEM((C, HK), f32),
          pltpu.VMEM((C, HK), f32),
          pltpu.VMEM((ZR, HK), f32),
          pltpu.VMEM((BPT,), jnp.int32),
          pltpu.VMEM((BPT,), jnp.int32),
          pltpu.VMEM((BPT,), f32),
          pltpu.VMEM_SHARED((N_NODES, HK), f32),
          pltpu.SemaphoreType.DMA,
          pltpu.SemaphoreType.DMA,
          pltpu.SemaphoreType.DMA,
      ],
  )
  return kern(ego_split, er4, ec4, ev4, user, item)


def kernel(Gu, Gi, edge_row, edge_col, edge_vals, user, item):
  ego = jnp.concatenate([Gu, Gi], axis=0)
  ego_split = jnp.stack([ego[:, :HK], ego[:, HK:]])
  er4 = edge_row.reshape(NT, NBLK, NB, C)
  ec4 = edge_col.reshape(NT, NBLK, NB, C)
  ev3 = edge_vals.reshape(NT, NBLK, NB * C)
  gu, gi, xui, _, _ = _run(ego_split, er4, ec4, ev3, user, item)
  gamma_u = jnp.concatenate([gu[0], gu[1]], axis=1)
  gamma_i = jnp.concatenate([gi[0], gi[1]], axis=1)
  return (xui[0] + xui[1], gamma_u, gamma_i)


# speedup vs baseline: 6.6062x; 1.3599x over previous
import jax
import jax.numpy as jnp
from jax import lax
from jax.experimental import pallas as pl
from jax.experimental.pallas import tpu as pltpu
from jax.experimental.pallas import tpu_sc as plsc

NUM_USERS = 5000
NUM_ITEMS = 5000
EMBED_K = 256
N_LAYERS = 3
N_EDGES = 160000
BATCH = 4096
N_NODES = NUM_USERS + NUM_ITEMS

NC = 2
NT = 16
HK = EMBED_K // NC
EPT = N_EDGES // NT
C = 125
NB = 20
NBLK = EPT // (C * NB)
BPT = BATCH // NT
GCH = 32
OCH = 64
NGC = BPT // GCH
Q = HK // 16
RCH = 128
NFULL = N_NODES // RCH
TAIL = N_NODES - NFULL * RCH
ZR = 16


def _body(ego, er4, ec4, ev3, user, item,
          gu_out, gi_out, xui_out, scr0, scr1, scr2,
          rowb, colb, valb, rows0, rows1, zbuf,
          ubuf, ibuf, xvm, accum, gsem, gsem1, ssem):
  gtmp = rows0
  gtmp2 = rows1
  c = lax.axis_index("c")
  s = lax.axis_index("s")

  def zero_row(r, _):
    for q in range(Q):
      zbuf[r, pl.ds(q * 16, 16)] = jnp.zeros((16,), jnp.float32)
    return 0
  lax.fori_loop(0, ZR, zero_row, 0)

  def zero_accum_chunk(m):
    base = m * RCH
    for h in range(RCH // ZR):
      pltpu.async_copy(zbuf, accum.at[pl.ds(base + h * ZR, ZR)], gsem)
    for h in range(RCH // ZR):
      pltpu.make_async_copy(zbuf, accum.at[pl.ds(base + h * ZR, ZR)],
                            gsem).wait()

  def tail_zero():
    pltpu.sync_copy(zbuf.at[pl.ds(0, TAIL)],
                    accum.at[pl.ds(NFULL * RCH, TAIL)])

  for i in range(5):
    m = s + NT * i
    @pl.when(m < NFULL)
    def _():
      zero_accum_chunk(m)
  @pl.when(s == NT - 1)
  def _():
    tail_zero()

  pltpu.sync_copy(user.at[pl.ds(s * BPT, BPT)], ubuf)
  pltpu.sync_copy(item.at[pl.ds(s * BPT, BPT)], ibuf)
  def shift_item(q, _):
    ibuf[pl.ds(q * 16, 16)] = ibuf[pl.ds(q * 16, 16)] + NUM_USERS
    return 0
  lax.fori_loop(0, BPT // 16, shift_item, 0)

  one = jnp.full((16,), 1, jnp.int32)

  def scale_chunk(buf, j):
    ev0 = jnp.full((16,), j * C, jnp.int32)
    @plsc.parallel_loop(0, C, step=1, unroll=4, carry=ev0)
    def _(e, ev):
      v = plsc.load_gather(valb, [ev])
      for q in range(Q):
        sl = pl.ds(q * 16, 16)
        buf[e, sl] = buf[e, sl] * v
      return ev + one

  srcs = (ego, scr0, scr1)
  dsts = (scr0, scr1, scr2)
  for k in range(1, N_LAYERS + 1):
    src = srcs[k - 1].at[c]
    dst = dsts[k - 1].at[c]
    plsc.subcore_barrier()

    def edge_block(b, _, src=src):
      pltpu.sync_copy(er4.at[s].at[b], rowb)
      pltpu.sync_copy(ec4.at[s].at[b], colb)
      pltpu.sync_copy(ev3.at[s].at[b], valb)

      pltpu.async_copy(src.at[colb.at[0]], rows0, gsem)

      def step(j, cur, nxt, sem_cur, sem_nxt):
        @pl.when(j >= 1)
        def _():
          pltpu.make_async_copy(nxt, accum.at[rowb.at[j - 1]], ssem).wait()
        @pl.when(j + 1 < NB)
        def _():
          pltpu.async_copy(src.at[colb.at[j + 1]], nxt, sem_nxt)
        pltpu.make_async_copy(src.at[colb.at[j]], cur, sem_cur).wait()
        scale_chunk(cur, j)
        pltpu.async_copy(cur, accum.at[rowb.at[j]], ssem, add=True)

      def pair(p, _):
        step(2 * p, rows0, rows1, gsem, gsem1)
        step(2 * p + 1, rows1, rows0, gsem1, gsem)
        return 0
      lax.fori_loop(0, NB // 2, pair, 0)
      pltpu.make_async_copy(rows1, accum.at[rowb.at[NB - 1]], ssem).wait()
      return 0
    with jax.named_scope(f"edges{k}"):
      lax.fori_loop(0, NBLK, edge_block, 0)
      plsc.subcore_barrier()

    with jax.named_scope(f"copyout{k}"):
      for i in range(5):
        m = s + NT * i
        @pl.when(m < NFULL)
        def _():
          for h, buf in ((0, rows0), (1, rows1)):
            rsl = pl.ds(m * RCH + h * OCH, OCH)
            pltpu.sync_copy(accum.at[rsl], buf.at[pl.ds(0, OCH)])
            pltpu.async_copy(buf.at[pl.ds(0, OCH)], dst.at[rsl], ssem)
          for h, buf in ((0, rows0), (1, rows1)):
            rsl = pl.ds(m * RCH + h * OCH, OCH)
            pltpu.make_async_copy(buf.at[pl.ds(0, OCH)], dst.at[rsl],
                                  ssem).wait()
          if k < N_LAYERS:
            zero_accum_chunk(m)
      @pl.when(s == NT - 1)
      def _():
        tsl = pl.ds(NFULL * RCH, TAIL)
        pltpu.sync_copy(accum.at[tsl], rows0.at[pl.ds(0, TAIL)])
        pltpu.sync_copy(rows0.at[pl.ds(0, TAIL)], dst.at[tsl])
        if k < N_LAYERS:
          tail_zero()

      plsc.subcore_barrier()

  FCH = 32
  sA = rows0.at[pl.ds(0, FCH)]
  sB = rows0.at[pl.ds(32, FCH)]
  sC = rows0.at[pl.ds(64, FCH)]
  sD = rows1.at[pl.ds(0, FCH)]
  sU = rows1.at[pl.ds(32, FCH)]
  AL = (0.25, 0.125, 1.0 / 3.0 / 4.0, 0.0625)

  def fire_gathers(idx_sl):
    pltpu.async_copy(ego.at[c].at[idx_sl], sA, gsem)
    pltpu.async_copy(scr0.at[c].at[idx_sl], sB, gsem)
    pltpu.async_copy(scr1.at[c].at[idx_sl], sC, gsem)
    pltpu.async_copy(scr2.at[c].at[idx_sl], sD, gsem)

  def drain_gathers(idx_sl):
    pltpu.make_async_copy(ego.at[c].at[idx_sl], sA, gsem).wait()
    pltpu.make_async_copy(scr0.at[c].at[idx_sl], sB, gsem).wait()
    pltpu.make_async_copy(scr1.at[c].at[idx_sl], sC, gsem).wait()
    pltpu.make_async_copy(scr2.at[c].at[idx_sl], sD, gsem).wait()

  def combine(dst_row_off):
    dref = rows1 if dst_row_off else rows0
    @plsc.parallel_loop(0, FCH, step=1, unroll=2)
    def _(r, *_a):
      for q in range(Q):
        sl = pl.ds(q * 16, 16)
        v = rows0[r, sl] * AL[0] + rows0[32 + r, sl] * AL[1]
        v = v + rows0[64 + r, sl] * AL[2] + rows1[r, sl] * AL[3]
        dref[dst_row_off + r, sl] = v

  with jax.named_scope("final"):
    for p in range(BPT // FCH):
      bsl = pl.ds(p * FCH, FCH)
      osl = pl.ds(s * BPT + p * FCH, FCH)
      uidx = ubuf.at[bsl]
      iidx = ibuf.at[bsl]
      if p > 0:
        pltpu.make_async_copy(sA, gi_out.at[c].at[_prev_osl], gsem1).wait()
      fire_gathers(uidx)
      if p > 0:
        pltpu.make_async_copy(sU, gu_out.at[c].at[_prev_osl], ssem).wait()
      drain_gathers(uidx)
      combine(32)
      pltpu.async_copy(sU, gu_out.at[c].at[osl], ssem)
      fire_gathers(iidx)
      drain_gathers(iidx)
      combine(0)
      def dot_group(g, _):
        irow = g * 16 + lax.iota(jnp.int32, 16)
        urow = 32 + g * 16 + lax.iota(jnp.int32, 16)
        def dot_dim(d, acc):
          col = jnp.full((16,), d, jnp.int32)
          u = plsc.load_gather(rows1, [urow, col])
          v = plsc.load_gather(rows0, [irow, col])
          return acc + u * v
        acc = lax.fori_loop(0, HK, dot_dim, jnp.zeros((16,), jnp.float32))
        xvm[pl.ds(p * FCH + g * 16, 16)] = acc
        return 0
      lax.fori_loop(0, FCH // 16, dot_group, 0)
      pltpu.async_copy(sA, gi_out.at[c].at[osl], gsem1)
      _prev_osl = osl
    pltpu.make_async_copy(sA, gi_out.at[c].at[_prev_osl], gsem1).wait()
    pltpu.make_async_copy(sU, gu_out.at[c].at[_prev_osl], ssem).wait()
    pltpu.sync_copy(xvm, xui_out.at[c].at[pl.ds(s * BPT, BPT)])


@jax.jit
def _run(ego_split, er4, ec4, ev4, user, item):
  f32 = jnp.float32
  kern = pl.kernel(
      _body,
      out_type=(
          jax.ShapeDtypeStruct((NC, BATCH, HK), f32),
          jax.ShapeDtypeStruct((NC, BATCH, HK), f32),
          jax.ShapeDtypeStruct((NC, BATCH), f32),
          jax.ShapeDtypeStruct((NC, N_NODES, HK), f32),
          jax.ShapeDtypeStruct((NC, N_NODES, HK), f32),
          jax.ShapeDtypeStruct((NC, N_NODES, HK), f32),
      ),
      mesh=plsc.VectorSubcoreMesh(core_axis_name="c", subcore_axis_name="s"),
      compiler_params=pltpu.CompilerParams(needs_layout_passes=False),
      scratch_types=[
          pltpu.VMEM((NB, C), jnp.int32),
          pltpu.VMEM((NB, C), jnp.int32),
          pltpu.VMEM((NB * C,), f32),
          pltpu.VMEM((C, HK), f32),
          pltpu.VMEM((C, HK), f32),
          pltpu.VMEM((ZR, HK), f32),
          pltpu.VMEM((BPT,), jnp.int32),
          pltpu.VMEM((BPT,), jnp.int32),
          pltpu.VMEM((BPT,), f32),
          pltpu.VMEM_SHARED((N_NODES, HK), f32),
          pltpu.SemaphoreType.DMA,
          pltpu.SemaphoreType.DMA,
          pltpu.SemaphoreType.DMA,
      ],
  )
  return kern(ego_split, er4, ec4, ev4, user, item)


def kernel(Gu, Gi, edge_row, edge_col, edge_vals, user, item):
  ego = jnp.concatenate([Gu, Gi], axis=0)
  ego_split = jnp.stack([ego[:, :HK], ego[:, HK:]])
  er4 = edge_row.reshape(NT, NBLK, NB, C)
  ec4 = edge_col.reshape(NT, NBLK, NB, C)
  ev3 = edge_vals.reshape(NT, NBLK, NB * C)
  gu, gi, xui, _, _, _ = _run(ego_split, er4, ec4, ev3, user, item)
  gamma_u = jnp.concatenate([gu[0], gu[1]], axis=1)
  gamma_i = jnp.concatenate([gi[0], gi[1]], axis=1)
  return (xui[0] + xui[1], gamma_u, gamma_i)
